# baseline (device time: 200499 ns/iter reference)
import jax
import jax.numpy as jnp
from jax import lax
from jax.experimental import pallas as pl
from jax.experimental.pallas import tpu as pltpu

N_DEV = 16
P = 4
Z = 4


def kernel(x):
    m_per, n = x.shape
    half = m_per // 2

    def body(x_ref, out_ref,
             zcw_s, zcw_r, zccw_s, zccw_r,
             pcwf_s, pcwf_r, pcwh_s, pcwh_r,
             pccwf_s, pccwf_r, pccwh_s, pccwh_r):
        my_pos = lax.axis_index("i")
        q = lax.rem(my_pos, P)
        zb = my_pos - q
        right = zb + lax.rem(q + 1, P)
        left = zb + lax.rem(q + 3, P)
        zup = lax.rem(my_pos + P, N_DEV)
        zdn = lax.rem(my_pos + (Z - 1) * P, N_DEV)

        def pos(dz, qq):
            return lax.rem(zb + P * dz, N_DEV) + qq

        def full_rows(p_):
            return pl.ds(p_ * m_per, m_per)

        def top_rows(p_):
            return pl.ds(p_ * m_per, half)

        def bot_rows(p_):
            return pl.ds(p_ * m_per + half, half)

        def desc(rows, ssem, rsem, target):
            return pltpu.make_async_remote_copy(
                src_ref=out_ref.at[rows, :],
                dst_ref=out_ref.at[rows, :],
                send_sem=ssem,
                recv_sem=rsem,
                device_id=(target,),
                device_id_type=pl.DeviceIdType.MESH,
            )

        barrier_sem = pltpu.get_barrier_semaphore()
        for nbr in (left, right, zup, zdn):
            pl.semaphore_signal(
                barrier_sem, inc=1,
                device_id=(nbr,), device_id_type=pl.DeviceIdType.MESH,
            )
        pl.semaphore_wait(barrier_sem, 4)

        out_ref[full_rows(my_pos), :] = x_ref[:, :]

        zcw0 = desc(full_rows(my_pos), zcw_s.at[0], zcw_r.at[0], zup)
        zcw1 = desc(top_rows(pos(Z - 1, q)), zcw_s.at[1], zcw_r.at[1], zup)
        zccw0 = desc(full_rows(my_pos), zccw_s.at[0], zccw_r.at[0], zdn)
        zccw1 = desc(bot_rows(pos(1, q)), zccw_s.at[1], zccw_r.at[1], zdn)

        ql = lax.rem(q + 3, P)
        qr = lax.rem(q + 1, P)
        pcw_f = [desc(full_rows(pos(dz, q)), pcwf_s.at[dz], pcwf_r.at[dz],
                      right) for dz in range(Z)]
        pcw_h = [desc(top_rows(pos(dz, ql)), pcwh_s.at[dz], pcwh_r.at[dz],
                      right) for dz in range(Z)]
        pccw_f = [desc(full_rows(pos(dz, q)), pccwf_s.at[dz], pccwf_r.at[dz],
                       left) for dz in range(Z)]
        pccw_h = [desc(bot_rows(pos(dz, qr)), pccwh_s.at[dz], pccwh_r.at[dz],
                       left) for dz in range(Z)]

        zcw0.start()
        zccw0.start()
        pcw_f[0].start()
        pccw_f[0].start()

        def after_zcw0():
            zcw1.start()
            pcw_f[Z - 1].start()
            pccw_f[Z - 1].start()

        def after_zccw0():
            zccw1.start()
            pcw_f[1].start()
            pccw_f[1].start()

        z0 = my_pos < P

        @pl.when(z0)
        def _():
            zccw0.wait_recv()
            after_zccw0()
            zcw0.wait_recv()
            after_zcw0()

        @pl.when(jnp.logical_not(z0))
        def _():
            zcw0.wait_recv()
            after_zcw0()
            zccw0.wait_recv()
            after_zccw0()

        pcw_f[0].wait_recv()
        pcw_h[0].start()
        pccw_f[0].wait_recv()
        pccw_h[0].start()
        zcw1.wait_recv()
        zccw1.wait_recv()
        pcw_f[2].start()
        pccw_f[2].start()

        def fwd(dz):
            pcw_f[dz].wait_recv()
            pcw_h[dz].start()
            pccw_f[dz].wait_recv()
            pccw_h[dz].start()

        @pl.when(z0)
        def _():
            for dz in (1, Z - 1, 2):
                fwd(dz)

        @pl.when(jnp.logical_not(z0))
        def _():
            for dz in (Z - 1, 1, 2):
                fwd(dz)
        for dz in range(Z):
            pcw_h[dz].wait_recv()
            pccw_h[dz].wait_recv()
        for d in (zcw0, zcw1, zccw0, zccw1, *pcw_f, *pcw_h,
                  *pccw_f, *pccw_h):
            d.wait_send()

    dma2 = pltpu.SemaphoreType.DMA((2,))
    dma4 = pltpu.SemaphoreType.DMA((Z,))
    return pl.pallas_call(
        body,
        out_shape=jax.ShapeDtypeStruct((N_DEV * m_per, n), x.dtype),
        in_specs=[pl.BlockSpec(memory_space=pltpu.VMEM)],
        out_specs=pl.BlockSpec(memory_space=pltpu.VMEM),
        scratch_shapes=[
            dma2, dma2, dma2, dma2,
            dma4, dma4, dma4, dma4,
            dma4, dma4, dma4, dma4,
        ],
        compiler_params=pltpu.CompilerParams(collective_id=0),
    )(x)


# device time: 180043 ns/iter; 1.1136x vs baseline; 1.1136x over previous
import jax
import jax.numpy as jnp
from jax import lax
from jax.experimental import pallas as pl
from jax.experimental.pallas import tpu as pltpu

N_DEV = 16
P = 4
Z = 4
T, B, H = 0, 1, 2


def kernel(x):
    m_per, n = x.shape
    half = m_per // 2

    def body(x_ref, out_ref,
             zcw_s, zcw_r, zccw_s, zccw_r,
             pcw_s, pcw_r, pccw_s, pccw_r):
        my_pos = lax.axis_index("i")
        q = lax.rem(my_pos, P)
        zb = my_pos - q
        right = zb + lax.rem(q + 1, P)
        left = zb + lax.rem(q + 3, P)
        ql = lax.rem(q + 3, P)
        qr = lax.rem(q + 1, P)
        zup = lax.rem(my_pos + P, N_DEV)
        zdn = lax.rem(my_pos + (Z - 1) * P, N_DEV)

        def pos(dz, qq):
            return lax.rem(zb + P * dz, N_DEV) + qq

        def top(p_):
            return pl.ds(p_ * m_per, half)

        def bot(p_):
            return pl.ds(p_ * m_per + half, half)

        def desc(rows, ssem, rsem, target):
            return pltpu.make_async_remote_copy(
                src_ref=out_ref.at[rows, :],
                dst_ref=out_ref.at[rows, :],
                send_sem=ssem,
                recv_sem=rsem,
                device_id=(target,),
                device_id_type=pl.DeviceIdType.MESH,
            )

        barrier_sem = pltpu.get_barrier_semaphore()
        for nbr in (left, right, zup, zdn):
            pl.semaphore_signal(
                barrier_sem, inc=1,
                device_id=(nbr,), device_id_type=pl.DeviceIdType.MESH,
            )
        pl.semaphore_wait(barrier_sem, 4)

        out_ref[pl.ds(my_pos * m_per, m_per), :] = x_ref[:, :]

        zcwT0 = desc(top(my_pos), zcw_s.at[0], zcw_r.at[0], zup)
        zcwB0 = desc(bot(my_pos), zcw_s.at[1], zcw_r.at[1], zup)
        zcw1 = desc(top(pos(Z - 1, q)), zcw_s.at[2], zcw_r.at[2], zup)
        zccwT0 = desc(top(my_pos), zccw_s.at[0], zccw_r.at[0], zdn)
        zccwB0 = desc(bot(my_pos), zccw_s.at[1], zccw_r.at[1], zdn)
        zccw1 = desc(bot(pos(1, q)), zccw_s.at[2], zccw_r.at[2], zdn)

        fT = [desc(top(pos(dz, q)), pcw_s.at[3 * dz + T],
                   pcw_r.at[3 * dz + T], right) for dz in range(Z)]
        fB = [desc(bot(pos(dz, q)), pcw_s.at[3 * dz + B],
                   pcw_r.at[3 * dz + B], right) for dz in range(Z)]
        h = [desc(top(pos(dz, ql)), pcw_s.at[3 * dz + H],
                  pcw_r.at[3 * dz + H], right) for dz in range(Z)]
        gT = [desc(top(pos(dz, q)), pccw_s.at[3 * dz + T],
                   pccw_r.at[3 * dz + T], left) for dz in range(Z)]
        gB = [desc(bot(pos(dz, q)), pccw_s.at[3 * dz + B],
                   pccw_r.at[3 * dz + B], left) for dz in range(Z)]
        hh = [desc(bot(pos(dz, qr)), pccw_s.at[3 * dz + H],
                   pccw_r.at[3 * dz + H], left) for dz in range(Z)]

        zcwT0.start()
        zcwB0.start()
        zccwT0.start()
        zccwB0.start()
        fT[0].start()
        fB[0].start()
        gT[0].start()
        gB[0].start()
        fT[0].wait_recv()
        h[0].start()
        gB[0].wait_recv()
        hh[0].start()
        zcwT0.wait_recv()
        zcw1.start()
        fT[Z - 1].start()
        gT[Z - 1].start()
        zcwB0.wait_recv()
        fB[Z - 1].start()
        gB[Z - 1].start()
        zccwT0.wait_recv()
        fT[1].start()
        gT[1].start()
        zccwB0.wait_recv()
        zccw1.start()
        fB[1].start()
        gB[1].start()
        zcw1.wait_recv()
        fT[2].start()
        gT[2].start()
        zccw1.wait_recv()
        fB[2].start()
        gB[2].start()
        for dz in (Z - 1, 1, 2):
            fT[dz].wait_recv()
            h[dz].start()
            gB[dz].wait_recv()
            hh[dz].start()
        for dz in range(Z):
            if dz:
                fB[dz].wait_recv()
                gT[dz].wait_recv()
            h[dz].wait_recv()
            hh[dz].wait_recv()
        fB[0].wait_recv()
        gT[0].wait_recv()
        for d in (zcwT0, zcwB0, zcw1, zccwT0, zccwB0, zccw1,
                  *fT, *fB, *h, *gT, *gB, *hh):
            d.wait_send()

    dma3 = pltpu.SemaphoreType.DMA((3,))
    dma12 = pltpu.SemaphoreType.DMA((3 * Z,))
    return pl.pallas_call(
        body,
        out_shape=jax.ShapeDtypeStruct((N_DEV * m_per, n), x.dtype),
        in_specs=[pl.BlockSpec(memory_space=pltpu.VMEM)],
        out_specs=pl.BlockSpec(memory_space=pltpu.VMEM),
        scratch_shapes=[
            dma3, dma3, dma3, dma3,
            dma12, dma12, dma12, dma12,
        ],
        compiler_params=pltpu.CompilerParams(collective_id=0),
    )(x)


# device time: 169713 ns/iter; 1.1814x vs baseline; 1.0609x over previous
import jax
import jax.numpy as jnp
from jax import lax
from jax.experimental import pallas as pl
from jax.experimental.pallas import tpu as pltpu

N_DEV = 16
P = 4
Z = 4
T, B, H = 0, 1, 2


def kernel(x):
    m_per, n = x.shape
    half = m_per // 2

    def body(x_ref, out_ref,
             zcw_s, zcw_r, zccw_s, zccw_r,
             pcw_s, pcw_r, pccw_s, pccw_r, own_sem):
        my_pos = lax.axis_index("i")
        q = lax.rem(my_pos, P)
        zb = my_pos - q
        right = zb + lax.rem(q + 1, P)
        left = zb + lax.rem(q + 3, P)
        ql = lax.rem(q + 3, P)
        qr = lax.rem(q + 1, P)
        zup = lax.rem(my_pos + P, N_DEV)
        zdn = lax.rem(my_pos + (Z - 1) * P, N_DEV)

        def pos(dz, qq):
            return lax.rem(zb + P * dz, N_DEV) + qq

        def top(p_):
            return pl.ds(p_ * m_per, half)

        def bot(p_):
            return pl.ds(p_ * m_per + half, half)

        def desc(rows, ssem, rsem, target):
            return pltpu.make_async_remote_copy(
                src_ref=out_ref.at[rows, :],
                dst_ref=out_ref.at[rows, :],
                send_sem=ssem,
                recv_sem=rsem,
                device_id=(target,),
                device_id_type=pl.DeviceIdType.MESH,
            )

        barrier_sem = pltpu.get_barrier_semaphore()
        for nbr in (left, right, zup, zdn):
            pl.semaphore_signal(
                barrier_sem, inc=1,
                device_id=(nbr,), device_id_type=pl.DeviceIdType.MESH,
            )
        pl.semaphore_wait(barrier_sem, 4)

        own_copy = pltpu.make_async_copy(
            x_ref, out_ref.at[pl.ds(my_pos * m_per, m_per), :], own_sem
        )
        own_copy.start()

        def xtop():
            return x_ref.at[pl.ds(0, half), :]

        def xbot():
            return x_ref.at[pl.ds(half, half), :]

        def xdesc(src, rows, ssem, rsem, target):
            return pltpu.make_async_remote_copy(
                src_ref=src,
                dst_ref=out_ref.at[rows, :],
                send_sem=ssem,
                recv_sem=rsem,
                device_id=(target,),
                device_id_type=pl.DeviceIdType.MESH,
            )

        zcwT0 = xdesc(xtop(), top(my_pos), zcw_s.at[0], zcw_r.at[0], zup)
        zcwB0 = xdesc(xbot(), bot(my_pos), zcw_s.at[1], zcw_r.at[1], zup)
        zcw1 = desc(top(pos(Z - 1, q)), zcw_s.at[2], zcw_r.at[2], zup)
        zccwT0 = xdesc(xtop(), top(my_pos), zccw_s.at[0], zccw_r.at[0], zdn)
        zccwB0 = xdesc(xbot(), bot(my_pos), zccw_s.at[1], zccw_r.at[1], zdn)
        zccw1 = desc(bot(pos(1, q)), zccw_s.at[2], zccw_r.at[2], zdn)

        def src_or_x(dz, topside):
            if dz == 0:
                return xtop() if topside else xbot()
            p_ = pos(dz, q)
            return out_ref.at[top(p_) if topside else bot(p_), :]

        fT = [xdesc(src_or_x(dz, True), top(pos(dz, q)),
                    pcw_s.at[3 * dz + T], pcw_r.at[3 * dz + T], right)
              for dz in range(Z)]
        fB = [xdesc(src_or_x(dz, False), bot(pos(dz, q)),
                    pcw_s.at[3 * dz + B], pcw_r.at[3 * dz + B], right)
              for dz in range(Z)]
        h = [desc(top(pos(dz, ql)), pcw_s.at[3 * dz + H],
                  pcw_r.at[3 * dz + H], right) for dz in range(Z)]
        gT = [xdesc(src_or_x(dz, True), top(pos(dz, q)),
                    pccw_s.at[3 * dz + T], pccw_r.at[3 * dz + T], left)
              for dz in range(Z)]
        gB = [xdesc(src_or_x(dz, False), bot(pos(dz, q)),
                    pccw_s.at[3 * dz + B], pccw_r.at[3 * dz + B], left)
              for dz in range(Z)]
        hh = [desc(bot(pos(dz, qr)), pccw_s.at[3 * dz + H],
                   pccw_r.at[3 * dz + H], left) for dz in range(Z)]

        zcwT0.start()
        zcwB0.start()
        zccwT0.start()
        zccwB0.start()
        fT[0].start()
        fB[0].start()
        gT[0].start()
        gB[0].start()
        fT[0].wait_recv()
        h[0].start()
        gB[0].wait_recv()
        hh[0].start()
        zcwT0.wait_recv()
        zcw1.start()
        fT[Z - 1].start()
        gT[Z - 1].start()
        zcwB0.wait_recv()
        fB[Z - 1].start()
        gB[Z - 1].start()
        zccwT0.wait_recv()
        fT[1].start()
        gT[1].start()
        zccwB0.wait_recv()
        zccw1.start()
        fB[1].start()
        gB[1].start()
        zcw1.wait_recv()
        fT[2].start()
        gT[2].start()
        zccw1.wait_recv()
        fB[2].start()
        gB[2].start()
        for dz in (Z - 1, 1, 2):
            fT[dz].wait_recv()
            h[dz].start()
            gB[dz].wait_recv()
            hh[dz].start()
        for dz in range(Z):
            if dz:
                fB[dz].wait_recv()
                gT[dz].wait_recv()
            h[dz].wait_recv()
            hh[dz].wait_recv()
        fB[0].wait_recv()
        gT[0].wait_recv()
        for d in (zcwT0, zcwB0, zcw1, zccwT0, zccwB0, zccw1,
                  *fT, *fB, *h, *gT, *gB, *hh):
            d.wait_send()
        own_copy.wait()

    dma3 = pltpu.SemaphoreType.DMA((3,))
    dma12 = pltpu.SemaphoreType.DMA((3 * Z,))
    return pl.pallas_call(
        body,
        out_shape=jax.ShapeDtypeStruct((N_DEV * m_per, n), x.dtype),
        in_specs=[pl.BlockSpec(memory_space=pltpu.VMEM)],
        out_specs=pl.BlockSpec(memory_space=pltpu.MemorySpace.HBM),
        scratch_shapes=[
            dma3, dma3, dma3, dma3,
            dma12, dma12, dma12, dma12,
            pltpu.SemaphoreType.DMA,
        ],
        compiler_params=pltpu.CompilerParams(collective_id=0),
    )(x)


# device time: 151411 ns/iter; 1.3242x vs baseline; 1.1209x over previous
import jax
import jax.numpy as jnp
from jax import lax
from jax.experimental import pallas as pl
from jax.experimental.pallas import tpu as pltpu

N_DEV = 16
P = 4
Z = 4
AH = 400
BR = 224
T, B, H = 0, 1, 2


def kernel(x):
    m_per, n = x.shape
    assert m_per == 2 * AH + BR

    def body(x_ref, out_ref,
             zcw_s, zcw_r, zccw_s, zccw_r,
             pcw_s, pcw_r, pccw_s, pccw_r,
             bp_s, bp_r, bz_s, bz_r, own_sem):
        my_pos = lax.axis_index("i")
        q = lax.rem(my_pos, P)
        zb = my_pos - q
        right = zb + lax.rem(q + 1, P)
        left = zb + lax.rem(q + 3, P)
        ql = lax.rem(q + 3, P)
        qr = lax.rem(q + 1, P)
        qo = lax.rem(q + 2, P)
        zup = lax.rem(my_pos + P, N_DEV)
        zdn = lax.rem(my_pos + (Z - 1) * P, N_DEV)

        def pos(dz, qq):
            return lax.rem(zb + P * dz, N_DEV) + qq

        def top(p_):
            return pl.ds(p_ * m_per, AH)

        def bot(p_):
            return pl.ds(p_ * m_per + AH, AH)

        def brow(p_):
            return pl.ds(p_ * m_per + 2 * AH, BR)

        def desc(rows, ssem, rsem, target):
            return pltpu.make_async_remote_copy(
                src_ref=out_ref.at[rows, :],
                dst_ref=out_ref.at[rows, :],
                send_sem=ssem,
                recv_sem=rsem,
                device_id=(target,),
                device_id_type=pl.DeviceIdType.MESH,
            )

        def xdesc(src, rows, ssem, rsem, target):
            return pltpu.make_async_remote_copy(
                src_ref=src,
                dst_ref=out_ref.at[rows, :],
                send_sem=ssem,
                recv_sem=rsem,
                device_id=(target,),
                device_id_type=pl.DeviceIdType.MESH,
            )

        xtop = x_ref.at[pl.ds(0, AH), :]
        xbot = x_ref.at[pl.ds(AH, AH), :]
        xb = x_ref.at[pl.ds(2 * AH, BR), :]

        barrier_sem = pltpu.get_barrier_semaphore()
        for nbr in (left, right, zup, zdn):
            pl.semaphore_signal(
                barrier_sem, inc=1,
                device_id=(nbr,), device_id_type=pl.DeviceIdType.MESH,
            )
        pl.semaphore_wait(barrier_sem, 4)

        own_copy = pltpu.make_async_copy(
            x_ref, out_ref.at[pl.ds(my_pos * m_per, m_per), :], own_sem
        )
        own_copy.start()

        zcwT0 = xdesc(xtop, top(my_pos), zcw_s.at[0], zcw_r.at[0], zup)
        zcwB0 = xdesc(xbot, bot(my_pos), zcw_s.at[1], zcw_r.at[1], zup)
        zcw1 = desc(top(pos(Z - 1, q)), zcw_s.at[2], zcw_r.at[2], zup)
        zccwT0 = xdesc(xtop, top(my_pos), zccw_s.at[0], zccw_r.at[0], zdn)
        zccwB0 = xdesc(xbot, bot(my_pos), zccw_s.at[1], zccw_r.at[1], zdn)
        zccw1 = desc(bot(pos(1, q)), zccw_s.at[2], zccw_r.at[2], zdn)

        def src_or_x(dz, topside):
            if dz == 0:
                return xtop if topside else xbot
            p_ = pos(dz, q)
            return out_ref.at[top(p_) if topside else bot(p_), :]

        fT = [xdesc(src_or_x(dz, True), top(pos(dz, q)),
                    pcw_s.at[3 * dz + T], pcw_r.at[3 * dz + T], right)
              for dz in range(Z)]
        fB = [xdesc(src_or_x(dz, False), bot(pos(dz, q)),
                    pcw_s.at[3 * dz + B], pcw_r.at[3 * dz + B], right)
              for dz in range(Z)]
        h = [desc(top(pos(dz, ql)), pcw_s.at[3 * dz + H],
                  pcw_r.at[3 * dz + H], right) for dz in range(Z)]
        gT = [xdesc(src_or_x(dz, True), top(pos(dz, q)),
                    pccw_s.at[3 * dz + T], pccw_r.at[3 * dz + T], left)
              for dz in range(Z)]
        gB = [xdesc(src_or_x(dz, False), bot(pos(dz, q)),
                    pccw_s.at[3 * dz + B], pccw_r.at[3 * dz + B], left)
              for dz in range(Z)]
        hh = [desc(bot(pos(dz, qr)), pccw_s.at[3 * dz + H],
                   pccw_r.at[3 * dz + H], left) for dz in range(Z)]

        bp_cw0 = xdesc(xb, brow(my_pos), bp_s.at[0], bp_r.at[0], right)
        bp_ccw0 = xdesc(xb, brow(my_pos), bp_s.at[1], bp_r.at[1], left)
        bp_fwd = desc(brow(pos(0, ql)), bp_s.at[2], bp_r.at[2], right)

        def bz_src(hop, qq):
            if hop == 0 and qq == 0:
                return xb
            return out_ref.at[brow(pos(Z - hop if hop else 0,
                                       lax.rem(q + qq, P))), :]

        bz = [[xdesc(bz_src(hop, qq),
                     brow(pos((Z - hop) % Z, lax.rem(q + qq, P))),
                     bz_s.at[4 * hop + qq], bz_r.at[4 * hop + qq], zup)
               for qq in range(P)] for hop in range(Z - 1)]

        zcwT0.start()
        zcwB0.start()
        zccwT0.start()
        zccwB0.start()
        fT[0].start()
        fB[0].start()
        gT[0].start()
        gB[0].start()
        bp_cw0.start()
        bp_ccw0.start()
        bz[0][0].start()
        bp_cw0.wait_recv()
        bp_fwd.start()
        bz[0][3].start()
        bp_ccw0.wait_recv()
        bz[0][1].start()
        bp_fwd.wait_recv()
        bz[0][2].start()
        fT[0].wait_recv()
        h[0].start()
        gB[0].wait_recv()
        hh[0].start()
        zcwT0.wait_recv()
        zcw1.start()
        fT[Z - 1].start()
        gT[Z - 1].start()
        zcwB0.wait_recv()
        fB[Z - 1].start()
        gB[Z - 1].start()
        zccwT0.wait_recv()
        fT[1].start()
        gT[1].start()
        zccwB0.wait_recv()
        zccw1.start()
        fB[1].start()
        gB[1].start()
        for qq in range(P):
            bz[0][qq].wait_recv()
            bz[1][qq].start()
        zcw1.wait_recv()
        fT[2].start()
        gT[2].start()
        zccw1.wait_recv()
        fB[2].start()
        gB[2].start()
        for dz in (Z - 1, 1, 2):
            fT[dz].wait_recv()
            h[dz].start()
            gB[dz].wait_recv()
            hh[dz].start()
        for qq in range(P):
            bz[1][qq].wait_recv()
            bz[2][qq].start()
        for dz in range(Z):
            if dz:
                fB[dz].wait_recv()
                gT[dz].wait_recv()
            h[dz].wait_recv()
            hh[dz].wait_recv()
        fB[0].wait_recv()
        gT[0].wait_recv()
        for qq in range(P):
            bz[2][qq].wait_recv()
        for d in (zcwT0, zcwB0, zcw1, zccwT0, zccwB0, zccw1,
                  *fT, *fB, *h, *gT, *gB, *hh,
                  bp_cw0, bp_ccw0, bp_fwd, *bz[0], *bz[1], *bz[2]):
            d.wait_send()
        own_copy.wait()

    dma3 = pltpu.SemaphoreType.DMA((3,))
    dma12 = pltpu.SemaphoreType.DMA((3 * Z,))
    return pl.pallas_call(
        body,
        out_shape=jax.ShapeDtypeStruct((N_DEV * m_per, n), x.dtype),
        in_specs=[pl.BlockSpec(memory_space=pltpu.VMEM)],
        out_specs=pl.BlockSpec(memory_space=pltpu.MemorySpace.HBM),
        scratch_shapes=[
            dma3, dma3, dma3, dma3,
            dma12, dma12, dma12, dma12,
            dma3, dma3,
            dma12, dma12,
            pltpu.SemaphoreType.DMA,
        ],
        compiler_params=pltpu.CompilerParams(collective_id=0),
    )(x)
